# Initial kernel scaffold; baseline (speedup 1.0000x reference)
import jax
import jax.numpy as jnp
from jax.experimental import pallas as pl


def kernel(hidden_states, mask_time_indices, W_proj, b_proj, codevectors):
    raise NotImplementedError("write your pallas kernel here")


# speedup vs baseline: 1.3184x; 1.3184x over previous
import functools

import jax
import jax.numpy as jnp
from jax import lax
from jax.experimental import pallas as pl
from jax.experimental.pallas import tpu as pltpu
from jax.experimental.pallas import tpu_sc as plsc

B, L, D = 8, 1024, 1024
G, V = 2, 320
GV = G * V
DG = 128
N = B * L
TM = 512
NEG = -1e30

NC, NS = 2, 16
NW = NC * NS
ROWS = N * G
BPW = ROWS // NW
CHUNK = 128
NCH = BPW // CHUNK


def _tc_body(x_ref, wt_ref, b_ref, m_ref, idx_ref, perp_ref, cnt_ref):
    i = pl.program_id(0)

    @pl.when(i == 0)
    def _init():
        cnt_ref[...] = jnp.zeros_like(cnt_ref)

    logits = (
        jnp.dot(x_ref[...], wt_ref[...], preferred_element_type=jnp.float32)
        + b_ref[0:1, :]
    )
    lanes = lax.broadcasted_iota(jnp.int32, (TM, GV), 1)
    in_g0 = lanes < V
    l0 = jnp.where(in_g0, logits, NEG)
    l1 = jnp.where(in_g0, NEG, logits)
    m0 = jnp.max(l0, axis=1, keepdims=True)
    m1 = jnp.max(l1, axis=1, keepdims=True)
    i0 = jnp.min(jnp.where(l0 >= m0, lanes, GV), axis=1, keepdims=True)
    i1 = jnp.min(jnp.where(l1 >= m1, lanes, GV), axis=1, keepdims=True)
    idx_ref[:, 0:1] = i0
    idx_ref[:, 1:2] = i1

    masked = m_ref[...] > 0.0
    i0c = jnp.where(masked, i0, GV + 1)
    i1c = jnp.where(masked, i1, GV + 1)
    hits = (lanes == i0c).astype(jnp.float32) + (lanes == i1c).astype(jnp.float32)
    cnt_ref[0:1, :] += jnp.sum(hits, axis=0, keepdims=True)

    @pl.when(i == pl.num_programs(0) - 1)
    def _fini():
        cnt = cnt_ref[0:1, :]
        denom = jnp.sum(cnt, keepdims=True) * 0.5
        avg = cnt / denom
        plogp = avg * jnp.log(avg + 1e-7)
        vlanes = lax.broadcasted_iota(jnp.int32, (1, GV), 1)
        s0 = jnp.sum(jnp.where(vlanes < V, plogp, 0.0), keepdims=True)
        s_all = jnp.sum(plogp, keepdims=True)
        perp_ref[...] = jnp.exp(-s0) + jnp.exp(-(s_all - s0))


def _tc_call(x, wt, b2d, mask_f):
    return pl.pallas_call(
        _tc_body,
        grid=(N // TM,),
        in_specs=[
            pl.BlockSpec((TM, D), lambda i: (i, 0)),
            pl.BlockSpec((D, GV), lambda i: (0, 0)),
            pl.BlockSpec((8, GV), lambda i: (0, 0)),
            pl.BlockSpec((TM, 1), lambda i: (i, 0)),
        ],
        out_specs=[
            pl.BlockSpec((TM, 2), lambda i: (i, 0)),
            pl.BlockSpec((1, 1), lambda i: (0, 0)),
        ],
        out_shape=[
            jax.ShapeDtypeStruct((N, 2), jnp.int32),
            jax.ShapeDtypeStruct((1, 1), jnp.float32),
        ],
        scratch_shapes=[pltpu.VMEM((8, GV), jnp.float32)],
    )(x, wt, b2d, mask_f)


def _sc_gather(table, idx2d):
    mesh = plsc.VectorSubcoreMesh(core_axis_name="c", subcore_axis_name="s")

    @functools.partial(
        pl.kernel,
        mesh=mesh,
        out_type=jax.ShapeDtypeStruct((ROWS, DG), jnp.float32),
        scratch_types=[
            pltpu.VMEM((NCH, CHUNK), jnp.int32),
            pltpu.VMEM((BPW, DG), jnp.float32),
            pltpu.SemaphoreType.DMA,
        ],
    )
    def gather_k(table_hbm, idx_hbm, out_hbm, idx_v, rows_v, sem):
        wid = lax.axis_index("s") * NC + lax.axis_index("c")
        base = wid * BPW
        pltpu.sync_copy(idx_hbm.at[pl.ds(wid * NCH, NCH)], idx_v)
        copies = []
        for j in range(NCH):
            copies.append(
                pltpu.async_copy(
                    table_hbm.at[idx_v.at[j]],
                    rows_v.at[pl.ds(j * CHUNK, CHUNK)],
                    sem,
                )
            )
        for c in copies:
            c.wait()
        pltpu.sync_copy(rows_v, out_hbm.at[pl.ds(base, BPW)])

    return gather_k(table, idx2d)


def kernel(hidden_states, mask_time_indices, W_proj, b_proj, codevectors):
    x = hidden_states.reshape(N, D)
    wt = W_proj.T
    b2d = jnp.broadcast_to(b_proj[None, :], (8, GV))
    mask_f = mask_time_indices.reshape(N, 1).astype(jnp.float32)

    idx, perp = _tc_call(x, wt, b2d, mask_f)

    table = codevectors.reshape(GV, DG)
    idx2d = idx.reshape(ROWS // CHUNK, CHUNK)
    rows = _sc_gather(table, idx2d)

    out = rows.reshape(B, L, G * DG)
    return out, perp.reshape(())


# speedup vs baseline: 1.4128x; 1.0716x over previous
import functools

import jax
import jax.numpy as jnp
from jax import lax
from jax.experimental import pallas as pl
from jax.experimental.pallas import tpu as pltpu
from jax.experimental.pallas import tpu_sc as plsc

B, L, D = 8, 1024, 1024
G, V = 2, 320
GV = G * V
VP = 384
GVP = G * VP
DG = 128
N = B * L
TM = 1024
RC = 256
NEG = -1e30

NC, NS = 2, 16
NW = NC * NS
TPW = N // NW
CHUNK = 128
NCH = TPW // CHUNK


def _tc_body(x_ref, wt_ref, b_ref, m_ref, idx0_ref, idx1_ref, perp_ref, cnt_ref):
    i = pl.program_id(0)

    @pl.when(i == 0)
    def _init():
        cnt_ref[...] = jnp.zeros_like(cnt_ref)

    lanes = lax.broadcasted_iota(jnp.int32, (RC, VP), 1)
    wt = wt_ref[...]
    bias = b_ref[0:1, :]
    for r in range(TM // RC):
        sl = pl.ds(r * RC, RC)
        logits = (
            jnp.dot(x_ref[sl, :], wt, preferred_element_type=jnp.float32)
            + bias
        )
        l0 = logits[:, :VP]
        l1 = logits[:, VP:]
        i0 = jnp.argmax(l0, axis=1).astype(jnp.int32)[:, None]
        i1 = jnp.argmax(l1, axis=1).astype(jnp.int32)[:, None]
        idx0_ref[sl, :] = i0
        idx1_ref[sl, :] = i1 + V

        mf = m_ref[sl, :]
        cnt_ref[0:1, :VP] += jnp.sum(jnp.where(lanes == i0, mf, 0.0), axis=0, keepdims=True)
        cnt_ref[0:1, VP:] += jnp.sum(jnp.where(lanes == i1, mf, 0.0), axis=0, keepdims=True)

    @pl.when(i == pl.num_programs(0) - 1)
    def _fini():
        cnt = cnt_ref[0:1, :]
        denom = jnp.sum(cnt, keepdims=True) * 0.5
        avg = cnt / denom
        plogp = avg * jnp.log(avg + 1e-7)
        vlanes = lax.broadcasted_iota(jnp.int32, (1, GVP), 1)
        s0 = jnp.sum(jnp.where(vlanes < VP, plogp, 0.0), keepdims=True)
        s_all = jnp.sum(plogp, keepdims=True)
        perp_ref[...] = jnp.exp(-s0) + jnp.exp(-(s_all - s0))


def _tc_call(x, wt, b2d, mask_f):
    return pl.pallas_call(
        _tc_body,
        grid=(N // TM,),
        in_specs=[
            pl.BlockSpec((TM, D), lambda i: (i, 0)),
            pl.BlockSpec((D, GVP), lambda i: (0, 0)),
            pl.BlockSpec((8, GVP), lambda i: (0, 0)),
            pl.BlockSpec((TM, 1), lambda i: (i, 0)),
        ],
        out_specs=[
            pl.BlockSpec((TM, 1), lambda i: (i, 0)),
            pl.BlockSpec((TM, 1), lambda i: (i, 0)),
            pl.BlockSpec((1, 1), lambda i: (0, 0)),
        ],
        out_shape=[
            jax.ShapeDtypeStruct((N, 1), jnp.int32),
            jax.ShapeDtypeStruct((N, 1), jnp.int32),
            jax.ShapeDtypeStruct((1, 1), jnp.float32),
        ],
        scratch_shapes=[pltpu.VMEM((8, GVP), jnp.float32)],
    )(x, wt, b2d, mask_f)


def _sc_gather(table, idx0_2d, idx1_2d):
    mesh = plsc.VectorSubcoreMesh(core_axis_name="c", subcore_axis_name="s")

    @functools.partial(
        pl.kernel,
        mesh=mesh,
        out_type=jax.ShapeDtypeStruct((N, G * DG), jnp.float32),
        scratch_types=[
            pltpu.VMEM((NCH, CHUNK), jnp.int32),
            pltpu.VMEM((NCH, CHUNK), jnp.int32),
            pltpu.VMEM((TPW, DG), jnp.float32),
            pltpu.VMEM((TPW, DG), jnp.float32),
            pltpu.SemaphoreType.DMA,
        ],
    )
    def gather_k(table_hbm, idx0_hbm, idx1_hbm, out_hbm, iv0, iv1, rows0, rows1, sem):
        wid = lax.axis_index("s") * NC + lax.axis_index("c")
        base = wid * TPW
        pltpu.sync_copy(idx0_hbm.at[pl.ds(wid * NCH, NCH)], iv0)
        pltpu.sync_copy(idx1_hbm.at[pl.ds(wid * NCH, NCH)], iv1)
        copies = []
        for j in range(NCH):
            dst = pl.ds(j * CHUNK, CHUNK)
            copies.append(pltpu.async_copy(table_hbm.at[iv0.at[j]], rows0.at[dst], sem))
            copies.append(pltpu.async_copy(table_hbm.at[iv1.at[j]], rows1.at[dst], sem))
        for c in copies:
            c.wait()
        pltpu.sync_copy(rows0, out_hbm.at[pl.ds(base, TPW), pl.ds(0, DG)])
        pltpu.sync_copy(rows1, out_hbm.at[pl.ds(base, TPW), pl.ds(DG, DG)])

    return gather_k(table, idx0_2d, idx1_2d)


def kernel(hidden_states, mask_time_indices, W_proj, b_proj, codevectors):
    x = hidden_states.reshape(N, D)
    wt = W_proj.T
    wt_pad = jnp.zeros((D, GVP), jnp.float32)
    wt_pad = lax.dynamic_update_slice(wt_pad, wt[:, :V], (0, 0))
    wt_pad = lax.dynamic_update_slice(wt_pad, wt[:, V:], (0, VP))
    b_pad = jnp.full((GVP,), NEG, jnp.float32)
    b_pad = lax.dynamic_update_slice(b_pad, b_proj[:V], (0,))
    b_pad = lax.dynamic_update_slice(b_pad, b_proj[V:], (VP,))
    b2d = jnp.broadcast_to(b_pad[None, :], (8, GVP))
    mask_f = mask_time_indices.reshape(N, 1).astype(jnp.float32)

    idx0, idx1, perp = _tc_call(x, wt_pad, b2d, mask_f)

    table = codevectors.reshape(GV, DG)
    idx0_2d = idx0.reshape(N // CHUNK, CHUNK)
    idx1_2d = idx1.reshape(N // CHUNK, CHUNK)
    out2d = _sc_gather(table, idx0_2d, idx1_2d)

    return out2d.reshape(B, L, G * DG), perp.reshape(())


# speedup vs baseline: 1.9589x; 1.3865x over previous
import functools

import jax
import jax.numpy as jnp
from jax import lax
from jax.experimental import pallas as pl
from jax.experimental.pallas import tpu as pltpu
from jax.experimental.pallas import tpu_sc as plsc

B, L, D = 8, 1024, 1024
G, V = 2, 320
GV = G * V
DG = 128
N = B * L
TM = 1024
RC = 256

NC, NS = 2, 16
NW = NC * NS
TPW = N // NW
CHUNK = 128
NCH = TPW // CHUNK

_DNUMS = (((1,), (1,)), ((), ()))


def _tc_body(x_ref, w_ref, b_ref, m_ref, idx0_ref, idx1_ref, perp_ref,
             cnt0_ref, cnt1_ref):
    i = pl.program_id(0)

    @pl.when(i == 0)
    def _init():
        cnt0_ref[...] = jnp.zeros_like(cnt0_ref)
        cnt1_ref[...] = jnp.zeros_like(cnt1_ref)

    lanes = lax.broadcasted_iota(jnp.int32, (RC, V), 1)
    w0 = w_ref[0:V, :]
    w1 = w_ref[V:GV, :]
    b0 = b_ref[0:1, 0:V]
    b1 = b_ref[0:1, V:GV]
    for r in range(TM // RC):
        sl = pl.ds(r * RC, RC)
        xc = x_ref[sl, :]
        l0 = lax.dot_general(xc, w0, _DNUMS, preferred_element_type=jnp.float32) + b0
        l1 = lax.dot_general(xc, w1, _DNUMS, preferred_element_type=jnp.float32) + b1
        i0 = jnp.argmax(l0, axis=1).astype(jnp.int32)
        i1 = jnp.argmax(l1, axis=1).astype(jnp.int32)
        idx0_ref[pl.ds(r * (RC // CHUNK), RC // CHUNK), :] = i0.reshape(RC // CHUNK, CHUNK)
        idx1_ref[pl.ds(r * (RC // CHUNK), RC // CHUNK), :] = (i1 + V).reshape(RC // CHUNK, CHUNK)

        mf = m_ref[sl, :]
        cnt0_ref[0:1, :] += jnp.sum(jnp.where(lanes == i0[:, None], mf, 0.0), axis=0, keepdims=True)
        cnt1_ref[0:1, :] += jnp.sum(jnp.where(lanes == i1[:, None], mf, 0.0), axis=0, keepdims=True)

    @pl.when(i == pl.num_programs(0) - 1)
    def _fini():
        c0 = cnt0_ref[0:1, :]
        c1 = cnt1_ref[0:1, :]
        denom = jnp.sum(c0, keepdims=True)
        a0 = c0 / denom
        a1 = c1 / denom
        p0 = jnp.sum(a0 * jnp.log(a0 + 1e-7), keepdims=True)
        p1 = jnp.sum(a1 * jnp.log(a1 + 1e-7), keepdims=True)
        perp_ref[...] = jnp.exp(-p0) + jnp.exp(-p1)


def _tc_call(x, w, b2d, mask_f):
    return pl.pallas_call(
        _tc_body,
        grid=(N // TM,),
        in_specs=[
            pl.BlockSpec((TM, D), lambda i: (i, 0)),
            pl.BlockSpec((GV, D), lambda i: (0, 0)),
            pl.BlockSpec((8, GV), lambda i: (0, 0)),
            pl.BlockSpec((TM, 1), lambda i: (i, 0)),
        ],
        out_specs=[
            pl.BlockSpec((TM // CHUNK, CHUNK), lambda i: (i, 0)),
            pl.BlockSpec((TM // CHUNK, CHUNK), lambda i: (i, 0)),
            pl.BlockSpec((1, 1), lambda i: (0, 0)),
        ],
        out_shape=[
            jax.ShapeDtypeStruct((N // CHUNK, CHUNK), jnp.int32),
            jax.ShapeDtypeStruct((N // CHUNK, CHUNK), jnp.int32),
            jax.ShapeDtypeStruct((1, 1), jnp.float32),
        ],
        scratch_shapes=[
            pltpu.VMEM((8, V), jnp.float32),
            pltpu.VMEM((8, V), jnp.float32),
        ],
    )(x, w, b2d, mask_f)


def _sc_gather(table, idx0_2d, idx1_2d):
    mesh = plsc.VectorSubcoreMesh(core_axis_name="c", subcore_axis_name="s")

    @functools.partial(
        pl.kernel,
        mesh=mesh,
        out_type=jax.ShapeDtypeStruct((N, G * DG), jnp.float32),
        scratch_types=[
            pltpu.VMEM((NCH, CHUNK), jnp.int32),
            pltpu.VMEM((NCH, CHUNK), jnp.int32),
            pltpu.VMEM((TPW, DG), jnp.float32),
            pltpu.VMEM((TPW, DG), jnp.float32),
            pltpu.SemaphoreType.DMA,
        ],
    )
    def gather_k(table_hbm, idx0_hbm, idx1_hbm, out_hbm, iv0, iv1, rows0, rows1, sem):
        wid = lax.axis_index("s") * NC + lax.axis_index("c")
        base = wid * TPW
        pltpu.sync_copy(idx0_hbm.at[pl.ds(wid * NCH, NCH)], iv0)
        pltpu.sync_copy(idx1_hbm.at[pl.ds(wid * NCH, NCH)], iv1)
        copies = []
        for j in range(NCH):
            dst = pl.ds(j * CHUNK, CHUNK)
            copies.append(pltpu.async_copy(table_hbm.at[iv0.at[j]], rows0.at[dst], sem))
            copies.append(pltpu.async_copy(table_hbm.at[iv1.at[j]], rows1.at[dst], sem))
        for c in copies:
            c.wait()
        pltpu.sync_copy(rows0, out_hbm.at[pl.ds(base, TPW), pl.ds(0, DG)])
        pltpu.sync_copy(rows1, out_hbm.at[pl.ds(base, TPW), pl.ds(DG, DG)])

    return gather_k(table, idx0_2d, idx1_2d)


def kernel(hidden_states, mask_time_indices, W_proj, b_proj, codevectors):
    x = hidden_states.reshape(N, D)
    b2d = jnp.broadcast_to(b_proj[None, :], (8, GV))
    mask_f = mask_time_indices.reshape(N, 1).astype(jnp.float32)

    idx0_2d, idx1_2d, perp = _tc_call(x, W_proj, b2d, mask_f)

    table = codevectors.reshape(GV, DG)
    out2d = _sc_gather(table, idx0_2d, idx1_2d)

    return out2d.reshape(B, L, G * DG), perp.reshape(())
